# Initial kernel scaffold; baseline (speedup 1.0000x reference)
import jax
import jax.numpy as jnp
from jax.experimental import pallas as pl


def kernel(x, Wr, W1, W2):
    raise NotImplementedError("write your pallas kernel here")


# speedup vs baseline: 1.5071x; 1.5071x over previous
import functools

import jax
import jax.numpy as jnp
from jax.experimental import pallas as pl
from jax.experimental.pallas import tpu as pltpu

E = 8
K = 2
AUX_COEFF = 0.01
Z_COEFF = 0.001
LANES = 128


def _router_body(x_ref, wr_ref, gate_ref, idsw_ref, aux_ref):
    x = x_ref[...]
    wr = wr_ref[...]
    logits = jax.lax.dot_general(
        x, wr, (((1,), (1,)), ((), ())), preferred_element_type=jnp.float32
    )
    n = logits.shape[0]
    lane = jax.lax.broadcasted_iota(jnp.int32, logits.shape, 1)
    valid = lane < E
    neg = jnp.float32(-1e30)
    logits = jnp.where(valid, logits, neg)
    m = jnp.max(logits, axis=1, keepdims=True)
    ex = jnp.where(valid, jnp.exp(logits - m), 0.0)
    den = jnp.sum(ex, axis=1, keepdims=True)
    probs = ex / den

    p0 = jnp.max(probs, axis=1, keepdims=True)
    a0 = jnp.min(jnp.where(probs == p0, lane, LANES), axis=1, keepdims=True)
    probs1 = jnp.where(lane == a0, -1.0, probs)
    p1 = jnp.max(probs1, axis=1, keepdims=True)
    a1 = jnp.min(jnp.where(probs1 == p1, lane, LANES), axis=1, keepdims=True)
    s = p0 + p1
    w0 = p0 / s
    w1 = p1 / s

    one0 = (lane == a0).astype(jnp.float32)
    one1 = (lane == a1).astype(jnp.float32)
    gate_ref[...] = one0 * w0 + one1 * w1

    idsw_ref[...] = jnp.where(
        lane == 0, a0.astype(jnp.float32),
        jnp.where(lane == 1, a1.astype(jnp.float32),
                  jnp.where(lane == 2, w0, jnp.where(lane == 3, w1, 0.0))))

    fraction = jnp.sum(one0, axis=0) / n
    mean_prob = jnp.sum(probs, axis=0) / n
    lbl = E * jnp.sum(fraction * mean_prob)
    lse = m[:, 0] + jnp.log(den[:, 0])
    z = jnp.sum(lse * lse) / n
    aux_ref[0, 0] = AUX_COEFF * lbl + Z_COEFF * z


def _router(x_flat, wr):
    n, d = x_flat.shape
    wr_pad = jnp.zeros((LANES, d), jnp.float32).at[:E].set(wr)
    gate, idsw, aux = pl.pallas_call(
        _router_body,
        out_shape=[
            jax.ShapeDtypeStruct((n, LANES), jnp.float32),
            jax.ShapeDtypeStruct((n, LANES), jnp.float32),
            jax.ShapeDtypeStruct((1, 1), jnp.float32),
        ],
        out_specs=[
            pl.BlockSpec((n, LANES), lambda: (0, 0)),
            pl.BlockSpec((n, LANES), lambda: (0, 0)),
            pl.BlockSpec(memory_space=pltpu.SMEM),
        ],
    )(x_flat, wr_pad)
    return gate, idsw, aux


def _dense_ffn_body(x_ref, w1_ref, w2_ref, gate_ref, out_ref):
    e = pl.program_id(1)
    fb = pl.program_id(2)

    @pl.when((e == 0) & (fb == 0))
    def _():
        out_ref[...] = jnp.zeros_like(out_ref)

    xb = x_ref[...]
    w1 = w1_ref[0]
    w2 = w2_ref[0]
    h = jax.lax.dot_general(
        xb, w1, (((1,), (1,)), ((), ())), preferred_element_type=jnp.float32)
    h = 0.5 * h * (1.0 + jax.lax.erf(h * 0.7071067811865476))
    y = jax.lax.dot_general(
        h, w2, (((1,), (1,)), ((), ())), preferred_element_type=jnp.float32)
    g = gate_ref[...]
    lane = jax.lax.broadcasted_iota(jnp.int32, g.shape, 1)
    ge = jnp.sum(jnp.where(lane == e, g, 0.0), axis=1, keepdims=True)
    out_ref[...] += ge * y


def _dense_ffn(x_flat, w1, w2, gate):
    n, d = x_flat.shape
    e, f, _ = w1.shape
    tn = 256
    fbs = 2048
    grid = (n // tn, e, f // fbs)
    return pl.pallas_call(
        _dense_ffn_body,
        grid=grid,
        in_specs=[
            pl.BlockSpec((tn, d), lambda t, ei, fi: (t, 0)),
            pl.BlockSpec((1, fbs, d), lambda t, ei, fi: (ei, fi, 0)),
            pl.BlockSpec((1, d, fbs), lambda t, ei, fi: (ei, 0, fi)),
            pl.BlockSpec((tn, LANES), lambda t, ei, fi: (t, 0)),
        ],
        out_specs=pl.BlockSpec((tn, d), lambda t, ei, fi: (t, 0)),
        out_shape=jax.ShapeDtypeStruct((n, d), jnp.float32),
    )(x_flat, w1, w2, gate)


def kernel(x, Wr, W1, W2):
    b, t, d = x.shape
    x_flat = x.reshape(b * t, d)
    gate, idsw, aux = _router(x_flat, Wr)
    out = _dense_ffn(x_flat, W1, W2, gate)
    return out.reshape(b, t, d), aux[0, 0]
